# baseline (speedup 1.0000x reference)
import functools

import jax
import jax.numpy as jnp
from jax import lax
from jax.experimental import pallas as pl
from jax.experimental.pallas import tpu as pltpu
from jax.experimental.pallas import tpu_sc as plsc

B = 4096
L = 200
D = 32
NC = 2
NS = 16
NW = NC * NS
BPW = B // NW
CB = 4
NCHUNK = BPW // CB
G0 = 128
G1 = L - G0


def _tc_split_idx(x):
    BR = 1024

    def body(x_ref, a_ref, b_ref):
        blk = x_ref[...]
        a_ref[...] = blk[:, :G0]
        b_ref[...] = jnp.pad(blk[:, G0:], ((0, 0), (0, 128 - G1)))

    return pl.pallas_call(
        body,
        grid=(B // BR,),
        in_specs=[pl.BlockSpec((BR, L), lambda i: (i, 0))],
        out_specs=[
            pl.BlockSpec((BR, 128), lambda i: (i, 0)),
            pl.BlockSpec((BR, 128), lambda i: (i, 0)),
        ],
        out_shape=[
            jax.ShapeDtypeStruct((B, 128), jnp.int32),
            jax.ShapeDtypeStruct((B, 128), jnp.int32),
        ],
    )(x)


def _sc_bag_sums(xa, xb, table):
    mesh = plsc.VectorSubcoreMesh(
        core_axis_name="c", subcore_axis_name="s", num_cores=NC, num_subcores=NS
    )

    @functools.partial(
        pl.kernel,
        out_type=jax.ShapeDtypeStruct((B, D), jnp.float32),
        mesh=mesh,
        compiler_params=pltpu.CompilerParams(use_tc_tiling_on_sc=False),
        scratch_types=[
            pltpu.VMEM((CB, 128), jnp.int32),
            pltpu.VMEM((CB, 128), jnp.int32),
            pltpu.VMEM((CB, 128), jnp.int32),
            pltpu.VMEM((CB, 128), jnp.int32),
            pltpu.VMEM((CB * L, D), jnp.float32),
            pltpu.VMEM((CB * L, D), jnp.float32),
            pltpu.VMEM((BPW, D), jnp.float32),
            pltpu.SemaphoreType.DMA,
            pltpu.SemaphoreType.DMA,
        ],
    )
    def k(xa_hbm, xb_hbm, tab_hbm, out_hbm,
          idxa0, idxb0, idxa1, idxb1, rows0, rows1, stage, sem0, sem1):
        wid = lax.axis_index("s") * NC + lax.axis_index("c")
        wbase = wid * BPW

        def fire(g, idxa, idxb, rows_ref, sem):
            bag0 = wbase + g * CB
            pltpu.sync_copy(xa_hbm.at[pl.ds(bag0, CB), :], idxa)
            pltpu.sync_copy(xb_hbm.at[pl.ds(bag0, CB), :], idxb)
            for bb in range(CB):
                pltpu.async_copy(
                    tab_hbm.at[idxa.at[bb, :]],
                    rows_ref.at[pl.ds(bb * L, G0), :], sem)
                pltpu.async_copy(
                    tab_hbm.at[idxb.at[bb, pl.ds(0, G1)]],
                    rows_ref.at[pl.ds(bb * L + G0, G1), :], sem)

        def drain(idxa, idxb, rows_ref, sem):
            for bb in range(CB):
                pltpu.make_async_copy(
                    tab_hbm.at[idxa.at[bb, :]],
                    rows_ref.at[pl.ds(bb * L, G0), :], sem).wait()
                pltpu.make_async_copy(
                    tab_hbm.at[idxb.at[bb, pl.ds(0, G1)]],
                    rows_ref.at[pl.ds(bb * L + G0, G1), :], sem).wait()

        def accum(g, rows_ref):
            for bb in range(CB):
                base = bb * L
                zz = jnp.zeros((16,), jnp.float32)

                def body(i, carry, base=base, rows_ref=rows_ref):
                    a0, a1, a2, a3, a4, a5, a6, a7 = carry
                    r = base + 4 * i
                    a0 = a0 + rows_ref[r, pl.ds(0, 16)]
                    a1 = a1 + rows_ref[r, pl.ds(16, 16)]
                    a2 = a2 + rows_ref[r + 1, pl.ds(0, 16)]
                    a3 = a3 + rows_ref[r + 1, pl.ds(16, 16)]
                    a4 = a4 + rows_ref[r + 2, pl.ds(0, 16)]
                    a5 = a5 + rows_ref[r + 2, pl.ds(16, 16)]
                    a6 = a6 + rows_ref[r + 3, pl.ds(0, 16)]
                    a7 = a7 + rows_ref[r + 3, pl.ds(16, 16)]
                    return (a0, a1, a2, a3, a4, a5, a6, a7)

                a = lax.fori_loop(0, L // 4, body, (zz,) * 8)
                bag = g * CB + bb
                stage[bag, pl.ds(0, 16)] = (a[0] + a[2]) + (a[4] + a[6])
                stage[bag, pl.ds(16, 16)] = (a[1] + a[3]) + (a[5] + a[7])

        fire(0, idxa0, idxb0, rows0, sem0)

        @pl.loop(0, NCHUNK, step=2)
        def _(g):
            fire(g + 1, idxa1, idxb1, rows1, sem1)
            drain(idxa0, idxb0, rows0, sem0)
            accum(g, rows0)

            @pl.when(g + 2 < NCHUNK)
            def _():
                fire(g + 2, idxa0, idxb0, rows0, sem0)

            drain(idxa1, idxb1, rows1, sem1)
            accum(g + 1, rows1)

        pltpu.sync_copy(stage, out_hbm.at[pl.ds(wbase, BPW), :])

    return k(xa, xb, table)


def _tc_head(x, summed, w, b2):
    C = w.shape[0]
    BT = 512

    def body(x_ref, s_ref, w_ref, b_ref, o_ref):
        cnt = jnp.sum((x_ref[...] != 0).astype(jnp.float32), axis=1, keepdims=True)
        denom = jnp.maximum(cnt, 1.0)
        acc = lax.dot_general(
            s_ref[...], w_ref[...], (((1,), (1,)), ((), ())),
            preferred_element_type=jnp.float32,
        )
        o_ref[...] = acc / denom + b_ref[...]

    return pl.pallas_call(
        body,
        grid=(B // BT,),
        in_specs=[
            pl.BlockSpec((BT, L), lambda i: (i, 0)),
            pl.BlockSpec((BT, D), lambda i: (i, 0)),
            pl.BlockSpec((C, D), lambda i: (0, 0)),
            pl.BlockSpec((1, C), lambda i: (0, 0)),
        ],
        out_specs=pl.BlockSpec((BT, C), lambda i: (i, 0)),
        out_shape=jax.ShapeDtypeStruct((B, C), jnp.float32),
    )(x, summed, w, b2)


def kernel(x, table, W, b):
    x = x.astype(jnp.int32)
    xa, xb = _tc_split_idx(x)
    summed = _sc_bag_sums(xa, xb, table)
    return _tc_head(x, summed, W, b.reshape(1, -1))


# speedup vs baseline: 1.6502x; 1.6502x over previous
import functools

import jax
import jax.numpy as jnp
from jax import lax
from jax.experimental import pallas as pl
from jax.experimental.pallas import tpu as pltpu
from jax.experimental.pallas import tpu_sc as plsc

B = 4096
L = 200
D = 32
C = 16
NC = 2
NS = 16
NW = NC * NS
BPW = B // NW
CB = 8
NCHUNK = BPW // CB
G0 = 128
G1 = L - G0

VOCAB = 1000000
PK = 128 // C
OB = 1024
BLK = PK * OB
NBLK = (VOCAB + BLK - 1) // BLK
VP = NBLK * BLK


def _tc_project_table(table, W):
    def body(t_ref, w_ref, o_ref):
        p = lax.dot_general(
            w_ref[...], t_ref[...], (((1,), (0,)), ((), ())),
            preferred_element_type=jnp.float32,
        )
        for c in range(PK):
            o_ref[:, c * C:(c + 1) * C] = p[:, c * OB:(c + 1) * OB].T

    packed = pl.pallas_call(
        body,
        grid=(NBLK,),
        in_specs=[
            pl.BlockSpec((D, BLK), lambda i: (0, i)),
            pl.BlockSpec((C, D), lambda i: (0, 0)),
        ],
        out_specs=pl.BlockSpec((OB, 128), lambda i: (i, 0)),
        out_shape=jax.ShapeDtypeStruct((VP // PK, 128), jnp.float32),
    )(table.T, W)
    return packed.reshape(VP, C)


def _translate_idx(v):
    return (
        (v & ~jnp.int32(8191))
        | ((v & jnp.int32(1023)) << 3)
        | ((v >> 10) & jnp.int32(7))
    )


def _tc_split_idx(x):
    BR = 1024

    def body(x_ref, a_ref, b_ref):
        blk = _translate_idx(x_ref[...])
        a_ref[...] = blk[:, :G0]
        b_ref[...] = jnp.pad(blk[:, G0:], ((0, 0), (0, 128 - G1)))

    return pl.pallas_call(
        body,
        grid=(B // BR,),
        in_specs=[pl.BlockSpec((BR, L), lambda i: (i, 0))],
        out_specs=[
            pl.BlockSpec((BR, 128), lambda i: (i, 0)),
            pl.BlockSpec((BR, 128), lambda i: (i, 0)),
        ],
        out_shape=[
            jax.ShapeDtypeStruct((B, 128), jnp.int32),
            jax.ShapeDtypeStruct((B, 128), jnp.int32),
        ],
    )(x)


def _sc_bag_sums(xa, xb, tabw):
    mesh = plsc.VectorSubcoreMesh(
        core_axis_name="c", subcore_axis_name="s", num_cores=NC, num_subcores=NS
    )

    @functools.partial(
        pl.kernel,
        out_type=jax.ShapeDtypeStruct((B, C), jnp.float32),
        mesh=mesh,
        compiler_params=pltpu.CompilerParams(use_tc_tiling_on_sc=False),
        scratch_types=[
            pltpu.VMEM((BPW, 128), jnp.int32),
            pltpu.VMEM((BPW, 128), jnp.int32),
            pltpu.VMEM((CB * L, C), jnp.float32),
            pltpu.VMEM((CB * L, C), jnp.float32),
            pltpu.VMEM((BPW, C), jnp.float32),
            pltpu.SemaphoreType.DMA,
            pltpu.SemaphoreType.DMA,
        ],
    )
    def k(xa_hbm, xb_hbm, tab_hbm, out_hbm,
          idxa, idxb, rows0, rows1, stage, sem0, sem1):
        wid = lax.axis_index("s") * NC + lax.axis_index("c")
        wbase = wid * BPW
        pltpu.sync_copy(xa_hbm.at[pl.ds(wbase, BPW), :], idxa)
        pltpu.sync_copy(xb_hbm.at[pl.ds(wbase, BPW), :], idxb)

        def fire(g, rows_ref, sem):
            for bb in range(CB):
                bag = g * CB + bb
                pltpu.async_copy(
                    tab_hbm.at[idxa.at[bag, :]],
                    rows_ref.at[pl.ds(bb * L, G0), :], sem)
                pltpu.async_copy(
                    tab_hbm.at[idxb.at[bag, pl.ds(0, G1)]],
                    rows_ref.at[pl.ds(bb * L + G0, G1), :], sem)

        def drain(g, rows_ref, sem):
            for bb in range(CB):
                bag = g * CB + bb
                pltpu.make_async_copy(
                    tab_hbm.at[idxa.at[bag, :]],
                    rows_ref.at[pl.ds(bb * L, G0), :], sem).wait()
                pltpu.make_async_copy(
                    tab_hbm.at[idxb.at[bag, pl.ds(0, G1)]],
                    rows_ref.at[pl.ds(bb * L + G0, G1), :], sem).wait()

        def accum(g, rows_ref):
            for bb in range(CB):
                base = bb * L
                zz = jnp.zeros((C,), jnp.float32)

                def body(i, carry, base=base, rows_ref=rows_ref):
                    a0, a1, a2, a3 = carry
                    r = base + 4 * i
                    a0 = a0 + rows_ref[r, pl.ds(0, C)]
                    a1 = a1 + rows_ref[r + 1, pl.ds(0, C)]
                    a2 = a2 + rows_ref[r + 2, pl.ds(0, C)]
                    a3 = a3 + rows_ref[r + 3, pl.ds(0, C)]
                    return (a0, a1, a2, a3)

                a = lax.fori_loop(0, L // 4, body, (zz,) * 4)
                bag = g * CB + bb
                stage[bag, pl.ds(0, C)] = (a[0] + a[1]) + (a[2] + a[3])

        fire(0, rows0, sem0)

        @pl.loop(0, NCHUNK, step=2)
        def _(g):
            fire(g + 1, rows1, sem1)
            drain(g, rows0, sem0)
            accum(g, rows0)

            @pl.when(g + 2 < NCHUNK)
            def _():
                fire(g + 2, rows0, sem0)

            drain(g + 1, rows1, sem1)
            accum(g + 1, rows1)

        pltpu.sync_copy(stage, out_hbm.at[pl.ds(wbase, BPW), :])

    return k(xa, xb, tabw)


def _tc_head(x, summed, b2):
    BT = 512

    def body(x_ref, s_ref, b_ref, o_ref):
        cnt = jnp.sum((x_ref[...] != 0).astype(jnp.float32), axis=1, keepdims=True)
        denom = jnp.maximum(cnt, 1.0)
        o_ref[...] = s_ref[...] / denom + b_ref[...]

    return pl.pallas_call(
        body,
        grid=(B // BT,),
        in_specs=[
            pl.BlockSpec((BT, L), lambda i: (i, 0)),
            pl.BlockSpec((BT, C), lambda i: (i, 0)),
            pl.BlockSpec((1, C), lambda i: (0, 0)),
        ],
        out_specs=pl.BlockSpec((BT, C), lambda i: (i, 0)),
        out_shape=jax.ShapeDtypeStruct((B, C), jnp.float32),
    )(x, summed, b2)


def kernel(x, table, W, b):
    x = x.astype(jnp.int32)
    xa, xb = _tc_split_idx(x)
    summed = _sc_bag_sums(xa, xb, _tc_project_table(table, W))
    return _tc_head(x, summed, b.reshape(1, -1))


# speedup vs baseline: 1.7843x; 1.0813x over previous
import functools

import jax
import jax.numpy as jnp
from jax import lax
from jax.experimental import pallas as pl
from jax.experimental.pallas import tpu as pltpu
from jax.experimental.pallas import tpu_sc as plsc

B = 4096
L = 200
D = 32
NC = 2
NS = 16
NW = NC * NS
BPW = B // NW
CB = 4
NCHUNK = BPW // CB
G0 = 128
G1 = L - G0


VOCAB = 1000000
OB = 4096
BLK = 4 * OB
NBLK = (VOCAB + BLK - 1) // BLK
VP = NBLK * BLK


def _tc_repack_table(table):
    def body(t_ref, o_ref):
        blk = t_ref[...]
        o_ref[...] = jnp.concatenate(
            [blk[:, c * OB:(c + 1) * OB].T for c in range(4)], axis=1)

    packed = pl.pallas_call(
        body,
        grid=(NBLK,),
        in_specs=[pl.BlockSpec((D, BLK), lambda i: (0, i))],
        out_specs=pl.BlockSpec((OB, 128), lambda i: (i, 0)),
        out_shape=jax.ShapeDtypeStruct((VP // 4, 128), jnp.float32),
    )(table.T)
    return packed.reshape(VP, D)


def _translate_idx(v):
    return (
        (v & ~jnp.int32(BLK - 1))
        | ((v & jnp.int32(OB - 1)) << 2)
        | ((v >> 12) & jnp.int32(3))
    )


def _tc_split_idx(x):
    BR = 1024

    def body(x_ref, a_ref, b_ref):
        blk = _translate_idx(x_ref[...])
        a_ref[...] = blk[:, :G0]
        b_ref[...] = jnp.pad(blk[:, G0:], ((0, 0), (0, 128 - G1)))

    return pl.pallas_call(
        body,
        grid=(B // BR,),
        in_specs=[pl.BlockSpec((BR, L), lambda i: (i, 0))],
        out_specs=[
            pl.BlockSpec((BR, 128), lambda i: (i, 0)),
            pl.BlockSpec((BR, 128), lambda i: (i, 0)),
        ],
        out_shape=[
            jax.ShapeDtypeStruct((B, 128), jnp.int32),
            jax.ShapeDtypeStruct((B, 128), jnp.int32),
        ],
    )(x)


def _sc_bag_sums(xa, xb, table):
    mesh = plsc.VectorSubcoreMesh(
        core_axis_name="c", subcore_axis_name="s", num_cores=NC, num_subcores=NS
    )

    @functools.partial(
        pl.kernel,
        out_type=jax.ShapeDtypeStruct((B, D), jnp.float32),
        mesh=mesh,
        compiler_params=pltpu.CompilerParams(use_tc_tiling_on_sc=False),
        scratch_types=[
            pltpu.VMEM((BPW, 128), jnp.int32),
            pltpu.VMEM((BPW, 128), jnp.int32),
            pltpu.VMEM((CB * L, D), jnp.float32),
            pltpu.VMEM((CB * L, D), jnp.float32),
            pltpu.VMEM((BPW, D), jnp.float32),
            pltpu.SemaphoreType.DMA,
            pltpu.SemaphoreType.DMA,
        ],
    )
    def k(xa_hbm, xb_hbm, tab_hbm, out_hbm,
          idxa, idxb, rows0, rows1, stage, sem0, sem1):
        wid = lax.axis_index("s") * NC + lax.axis_index("c")
        wbase = wid * BPW
        pltpu.sync_copy(xa_hbm.at[pl.ds(wbase, BPW), :], idxa)
        pltpu.sync_copy(xb_hbm.at[pl.ds(wbase, BPW), :], idxb)

        def fire(g, rows_ref, sem):
            for bb in range(CB):
                bag = g * CB + bb
                pltpu.async_copy(
                    tab_hbm.at[idxa.at[bag, :]],
                    rows_ref.at[pl.ds(bb * L, G0), :], sem)
                pltpu.async_copy(
                    tab_hbm.at[idxb.at[bag, pl.ds(0, G1)]],
                    rows_ref.at[pl.ds(bb * L + G0, G1), :], sem)

        def drain(g, rows_ref, sem):
            for bb in range(CB):
                bag = g * CB + bb
                pltpu.make_async_copy(
                    tab_hbm.at[idxa.at[bag, :]],
                    rows_ref.at[pl.ds(bb * L, G0), :], sem).wait()
                pltpu.make_async_copy(
                    tab_hbm.at[idxb.at[bag, pl.ds(0, G1)]],
                    rows_ref.at[pl.ds(bb * L + G0, G1), :], sem).wait()

        def accum(g, rows_ref):
            for bb in range(CB):
                base = bb * L
                zz = jnp.zeros((16,), jnp.float32)

                def body(i, carry, base=base, rows_ref=rows_ref):
                    a0, a1, a2, a3, a4, a5, a6, a7 = carry
                    r = base + 4 * i
                    a0 = a0 + rows_ref[r, pl.ds(0, 16)]
                    a1 = a1 + rows_ref[r, pl.ds(16, 16)]
                    a2 = a2 + rows_ref[r + 1, pl.ds(0, 16)]
                    a3 = a3 + rows_ref[r + 1, pl.ds(16, 16)]
                    a4 = a4 + rows_ref[r + 2, pl.ds(0, 16)]
                    a5 = a5 + rows_ref[r + 2, pl.ds(16, 16)]
                    a6 = a6 + rows_ref[r + 3, pl.ds(0, 16)]
                    a7 = a7 + rows_ref[r + 3, pl.ds(16, 16)]
                    return (a0, a1, a2, a3, a4, a5, a6, a7)

                a = lax.fori_loop(0, L // 4, body, (zz,) * 8)
                bag = g * CB + bb
                stage[bag, pl.ds(0, 16)] = (a[0] + a[2]) + (a[4] + a[6])
                stage[bag, pl.ds(16, 16)] = (a[1] + a[3]) + (a[5] + a[7])

        fire(0, rows0, sem0)

        @pl.loop(0, NCHUNK, step=2)
        def _(g):
            fire(g + 1, rows1, sem1)
            drain(g, rows0, sem0)
            accum(g, rows0)

            @pl.when(g + 2 < NCHUNK)
            def _():
                fire(g + 2, rows0, sem0)

            drain(g + 1, rows1, sem1)
            accum(g + 1, rows1)

        pltpu.sync_copy(stage, out_hbm.at[pl.ds(wbase, BPW), :])

    return k(xa, xb, table)


def _tc_head(x, summed, w, b2):
    C = w.shape[0]
    BT = 512

    def body(x_ref, s_ref, w_ref, b_ref, o_ref):
        cnt = jnp.sum((x_ref[...] != 0).astype(jnp.float32), axis=1, keepdims=True)
        denom = jnp.maximum(cnt, 1.0)
        acc = lax.dot_general(
            s_ref[...], w_ref[...], (((1,), (1,)), ((), ())),
            preferred_element_type=jnp.float32,
        )
        o_ref[...] = acc / denom + b_ref[...]

    return pl.pallas_call(
        body,
        grid=(B // BT,),
        in_specs=[
            pl.BlockSpec((BT, L), lambda i: (i, 0)),
            pl.BlockSpec((BT, D), lambda i: (i, 0)),
            pl.BlockSpec((C, D), lambda i: (0, 0)),
            pl.BlockSpec((1, C), lambda i: (0, 0)),
        ],
        out_specs=pl.BlockSpec((BT, C), lambda i: (i, 0)),
        out_shape=jax.ShapeDtypeStruct((B, C), jnp.float32),
    )(x, summed, w, b2)


def kernel(x, table, W, b):
    x = x.astype(jnp.int32)
    xa, xb = _tc_split_idx(x)
    summed = _sc_bag_sums(xa, xb, _tc_repack_table(table))
    return _tc_head(x, summed, W, b.reshape(1, -1))


# speedup vs baseline: 1.7860x; 1.0009x over previous
import functools

import jax
import jax.numpy as jnp
from jax import lax
from jax.experimental import pallas as pl
from jax.experimental.pallas import tpu as pltpu
from jax.experimental.pallas import tpu_sc as plsc

B = 4096
L = 200
D = 32
NC = 2
NS = 16
NW = NC * NS
BPW = B // NW
CB = 4
NCHUNK = BPW // CB
G0 = 128
G1 = L - G0


VOCAB = 1000000
OB = 4096
BLK = 4 * OB
NBLK = (VOCAB + BLK - 1) // BLK
VP = NBLK * BLK


def _tc_repack_table(table):
    def body(t_ref, o_ref):
        blk = t_ref[...]
        o_ref[...] = jnp.concatenate(
            [blk[:, c * OB:(c + 1) * OB].T for c in range(4)], axis=1)

    packed = pl.pallas_call(
        body,
        grid=(NBLK,),
        in_specs=[pl.BlockSpec((D, BLK), lambda i: (0, i))],
        out_specs=pl.BlockSpec((OB, 128), lambda i: (i, 0)),
        out_shape=jax.ShapeDtypeStruct((VP // 4, 128), jnp.float32),
    )(table.T)
    return packed.reshape(VP, D)


def _translate_idx(v):
    return (
        (v & ~jnp.int32(BLK - 1))
        | ((v & jnp.int32(OB - 1)) << 2)
        | ((v >> 12) & jnp.int32(3))
    )


def _tc_split_idx(x):
    BR = 1024

    def body(x_ref, a_ref, b_ref):
        blk = _translate_idx(x_ref[...])
        a_ref[...] = blk[:, :G0]
        b_ref[...] = jnp.pad(blk[:, G0:], ((0, 0), (0, 128 - G1)))

    return pl.pallas_call(
        body,
        grid=(B // BR,),
        in_specs=[pl.BlockSpec((BR, L), lambda i: (i, 0))],
        out_specs=[
            pl.BlockSpec((BR, 128), lambda i: (i, 0)),
            pl.BlockSpec((BR, 128), lambda i: (i, 0)),
        ],
        out_shape=[
            jax.ShapeDtypeStruct((B, 128), jnp.int32),
            jax.ShapeDtypeStruct((B, 128), jnp.int32),
        ],
    )(x)


def _sc_bag_sums(xa, xb, table):
    mesh = plsc.VectorSubcoreMesh(
        core_axis_name="c", subcore_axis_name="s", num_cores=NC, num_subcores=NS
    )

    @functools.partial(
        pl.kernel,
        out_type=jax.ShapeDtypeStruct((B, D), jnp.float32),
        mesh=mesh,
        compiler_params=pltpu.CompilerParams(use_tc_tiling_on_sc=False),
        scratch_types=[
            pltpu.VMEM((BPW, 128), jnp.int32),
            pltpu.VMEM((BPW, 128), jnp.int32),
            pltpu.VMEM((CB * L, D), jnp.float32),
            pltpu.VMEM((CB * L, D), jnp.float32),
            pltpu.VMEM((BPW, D), jnp.float32),
            pltpu.SemaphoreType.DMA,
            pltpu.SemaphoreType.DMA,
        ],
    )
    def k(xa_hbm, xb_hbm, tab_hbm, out_hbm,
          idxa, idxb, rows0, rows1, stage, sem0, sem1):
        wid = lax.axis_index("s") * NC + lax.axis_index("c")
        wbase = wid * BPW
        pltpu.sync_copy(xa_hbm.at[pl.ds(wbase, BPW), :], idxa)
        pltpu.sync_copy(xb_hbm.at[pl.ds(wbase, BPW), :], idxb)

        def fire(g, rows_ref, sem):
            for bb in range(CB):
                bag = g * CB + bb
                pltpu.async_copy(
                    tab_hbm.at[idxa.at[bag, :]],
                    rows_ref.at[pl.ds(bb * L, G0), :], sem)
                pltpu.async_copy(
                    tab_hbm.at[idxb.at[bag, pl.ds(0, G1)]],
                    rows_ref.at[pl.ds(bb * L + G0, G1), :], sem)

        def drain(g, rows_ref, sem):
            for bb in range(CB):
                bag = g * CB + bb
                pltpu.make_async_copy(
                    tab_hbm.at[idxa.at[bag, :]],
                    rows_ref.at[pl.ds(bb * L, G0), :], sem).wait()
                pltpu.make_async_copy(
                    tab_hbm.at[idxb.at[bag, pl.ds(0, G1)]],
                    rows_ref.at[pl.ds(bb * L + G0, G1), :], sem).wait()

        def accum(g, rows_ref):
            for bb in range(CB):
                base = bb * L
                zz = jnp.zeros((16,), jnp.float32)

                def body(i, carry, base=base, rows_ref=rows_ref):
                    a = list(carry)
                    r = base + 8 * i
                    for k in range(8):
                        a[(2 * k) % 8] = a[(2 * k) % 8] + rows_ref[r + k, pl.ds(0, 16)]
                        a[(2 * k + 1) % 8] = a[(2 * k + 1) % 8] + rows_ref[r + k, pl.ds(16, 16)]
                    return tuple(a)

                a = lax.fori_loop(0, L // 8, body, (zz,) * 8)
                bag = g * CB + bb
                stage[bag, pl.ds(0, 16)] = (a[0] + a[2]) + (a[4] + a[6])
                stage[bag, pl.ds(16, 16)] = (a[1] + a[3]) + (a[5] + a[7])

        fire(0, rows0, sem0)

        @pl.loop(0, NCHUNK, step=2)
        def _(g):
            fire(g + 1, rows1, sem1)
            drain(g, rows0, sem0)
            accum(g, rows0)

            @pl.when(g + 2 < NCHUNK)
            def _():
                fire(g + 2, rows0, sem0)

            drain(g + 1, rows1, sem1)
            accum(g + 1, rows1)

        pltpu.sync_copy(stage, out_hbm.at[pl.ds(wbase, BPW), :])

    return k(xa, xb, table)


def _tc_head(x, summed, w, b2):
    C = w.shape[0]
    BT = 512

    def body(x_ref, s_ref, w_ref, b_ref, o_ref):
        cnt = jnp.sum((x_ref[...] != 0).astype(jnp.float32), axis=1, keepdims=True)
        denom = jnp.maximum(cnt, 1.0)
        acc = lax.dot_general(
            s_ref[...], w_ref[...], (((1,), (1,)), ((), ())),
            preferred_element_type=jnp.float32,
        )
        o_ref[...] = acc / denom + b_ref[...]

    return pl.pallas_call(
        body,
        grid=(B // BT,),
        in_specs=[
            pl.BlockSpec((BT, L), lambda i: (i, 0)),
            pl.BlockSpec((BT, D), lambda i: (i, 0)),
            pl.BlockSpec((C, D), lambda i: (0, 0)),
            pl.BlockSpec((1, C), lambda i: (0, 0)),
        ],
        out_specs=pl.BlockSpec((BT, C), lambda i: (i, 0)),
        out_shape=jax.ShapeDtypeStruct((B, C), jnp.float32),
    )(x, summed, w, b2)


def kernel(x, table, W, b):
    x = x.astype(jnp.int32)
    xa, xb = _tc_split_idx(x)
    summed = _sc_bag_sums(xa, xb, _tc_repack_table(table))
    return _tc_head(x, summed, W, b.reshape(1, -1))
